# baseline (speedup 1.0000x reference)
import functools

import jax
import jax.numpy as jnp
from jax import lax
from jax.experimental import pallas as pl
from jax.experimental.pallas import tpu as pltpu
from jax.experimental.pallas import tpu_sc as plsc

N = 10000
D = 128
H_DIM = 256
E = 320000

NC, NS, L = 2, 16, 16
NW = NC * NS
HALF = 5120
JUNK = HALF
AGGH = 5248
RPT = AGGH // NS
HROWS = 2 * HALF
HSPT = HALF // NS
CHUNK = 64
CAPC = 160
CAP = CAPC * CHUNK
EPAD = NW * CAP
NR = 4
A = 3
S = 1
P = 2 * NR
U = P - S
IDX_BITS = 14
IDX_MASK = (1 << IDX_BITS) - 1
JUNKP = JUNK << IDX_BITS

_MESH = plsc.VectorSubcoreMesh(
    core_axis_name="c", subcore_axis_name="s", num_cores=NC, num_subcores=NS
)


def _sc_aggregate(h2, lists, cnts, zrows):
    @functools.partial(
        pl.kernel,
        mesh=_MESH,
        out_type=jax.ShapeDtypeStruct((NC, AGGH, D), jnp.float32),
        scratch_types=[
            pltpu.VMEM((CAP // 128, 128), jnp.int32),
            pltpu.VMEM((4, 16), jnp.int32),
            pltpu.VMEM((P, CHUNK), jnp.int32),
            pltpu.VMEM((P, CHUNK), jnp.int32),
            pltpu.VMEM((NR, CHUNK, D), jnp.float32),
            pltpu.VMEM_SHARED((HALF, D), jnp.float32),
            pltpu.VMEM_SHARED((AGGH, D), jnp.float32),
            pltpu.SemaphoreType.DMA,
            pltpu.SemaphoreType.DMA,
        ],
    )
    def body(h_hbm, lists_hbm, cnt_hbm, z_hbm, out_hbm,
             pk_v, cnt_v, sidx_v, didx_v, rows_v, h_sh, agg_sh, gsem, ssem):
        cid = lax.axis_index("c")
        sid = lax.axis_index("s")
        pltpu.sync_copy(z_hbm, agg_sh.at[pl.ds(sid * RPT, RPT)])
        pltpu.sync_copy(cnt_hbm, cnt_v)

        def unpack(c, k):
            slot = k % P
            for v in range(CHUNK // L):
                p = pk_v[c // 2, pl.ds((c % 2) * CHUNK + v * L, L)]
                sidx_v[slot, pl.ds(v * L, L)] = jnp.bitwise_and(p, IDX_MASK)
                didx_v[slot, pl.ds(v * L, L)] = jnp.bitwise_and(
                    lax.shift_right_logical(p, IDX_BITS), IDX_MASK
                )

        def step(c, k, wait_scat, do_unpack, do_gather):
            b = k % NR
            pltpu.make_async_copy(h_sh.at[sidx_v.at[k % P]], rows_v.at[b], gsem).wait()
            pltpu.async_copy(rows_v.at[b], agg_sh.at[didx_v.at[k % P]], ssem, add=True)
            if wait_scat:
                pltpu.make_async_copy(
                    rows_v.at[(k - S) % NR], agg_sh.at[didx_v.at[(k - S) % P]], ssem
                ).wait()
            if do_unpack:
                unpack(c + U, k + U)
            if do_gather:
                pltpu.async_copy(
                    h_sh.at[sidx_v.at[(k + A) % P]], rows_v.at[(k + A) % NR], gsem
                )

        def run_list(tc):
            for c in range(U):
                unpack(c, c)
            for b in range(A):
                pltpu.async_copy(h_sh.at[sidx_v.at[b]], rows_v.at[b], gsem)
            for k in range(P):
                step(k, k, k >= S, True, True)

            @pl.loop(P, tc - P, step=P)
            def _(g):
                for k in range(P):
                    step(g + k, k, True, True, True)

            base_c = tc - P
            for k in range(P):
                step(base_c + k, k, True, k < S, k < P - A)
            for k in range(P - S, P):
                pltpu.make_async_copy(
                    rows_v.at[k % NR], agg_sh.at[didx_v.at[k]], ssem
                ).wait()

        for s in (0, 1):
            pltpu.sync_copy(
                h_hbm.at[pl.ds(s * HALF + sid * HSPT, HSPT)],
                h_sh.at[pl.ds(sid * HSPT, HSPT)],
            )
            plsc.subcore_barrier()
            g = s * 2 + cid
            tcv = cnt_v[g, pl.ds(0, L)]
            tch = tcv[0]
            for li in (0, 1):
                row0 = pl.multiple_of(
                    sid * tch + li * lax.shift_right_logical(tch, 1), 8
                )
                pltpu.sync_copy(
                    lists_hbm.at[g].at[pl.ds(row0, CAP // 128)], pk_v
                )
                run_list(tch)
            plsc.subcore_barrier()

        pltpu.sync_copy(
            agg_sh.at[pl.ds(sid * RPT, RPT)],
            out_hbm.at[cid].at[pl.ds(sid * RPT, RPT)],
        )

    return body(h2, lists, cnts, zrows)


def _mlp_block(scale_ref, h_ref, a_ref, w1_ref, b1_ref, w2_ref, b2_ref, o_ref):
    z = scale_ref[0] * h_ref[...] + a_ref[...]
    z = jnp.maximum(
        jnp.dot(z, w1_ref[...], preferred_element_type=jnp.float32) + b1_ref[...], 0.0
    )
    z = jnp.dot(z, w2_ref[...], preferred_element_type=jnp.float32) + b2_ref[...]
    o_ref[...] = jnp.maximum(z, 0.0)


def _tc_mlp(h, a, scale, W1, b1, W2, b2):
    R = 1000
    grid = (N // R,)
    return pl.pallas_call(
        _mlp_block,
        grid=grid,
        in_specs=[
            pl.BlockSpec(memory_space=pltpu.SMEM),
            pl.BlockSpec((R, D), lambda i: (i, 0)),
            pl.BlockSpec((R, D), lambda i: (i, 0)),
            pl.BlockSpec((D, H_DIM), lambda i: (0, 0)),
            pl.BlockSpec((1, H_DIM), lambda i: (0, 0)),
            pl.BlockSpec((H_DIM, D), lambda i: (0, 0)),
            pl.BlockSpec((1, D), lambda i: (0, 0)),
        ],
        out_specs=pl.BlockSpec((R, D), lambda i: (i, 0)),
        out_shape=jax.ShapeDtypeStruct((N, D), jnp.float32),
    )(scale, h, a, W1, b1.reshape(1, H_DIM), W2, b2.reshape(1, D))


def kernel(x, edge_index, batch,
           eps0, W1_0, b1_0, W2_0, b2_0,
           eps1, W1_1, b1_1, W2_1, b2_1,
           eps2, W1_2, b1_2, W2_2, b2_2):
    src = edge_index[0]
    dst = edge_index[1]
    s_bit = (src >= HALF).astype(jnp.int32)
    d_bit = (dst >= HALF).astype(jnp.int32)
    p = (
        lax.shift_left(s_bit, 2 * IDX_BITS + 1)
        | lax.shift_left(d_bit, 2 * IDX_BITS)
        | lax.shift_left(dst - d_bit * HALF, IDX_BITS)
        | (src - s_bit * HALF)
    )
    grp = s_bit * 2 + d_bit
    perm = jnp.argsort(grp, stable=True)
    sorted_p = jnp.concatenate([p[perm], jnp.full((1,), JUNKP, jnp.int32)])
    counts = jnp.bincount(grp, length=4).astype(jnp.int32)
    start = jnp.concatenate(
        [jnp.zeros((1,), jnp.int32), jnp.cumsum(counts)[:-1].astype(jnp.int32)]
    )
    tiles_chunks = (counts + (NS * CHUNK - 1)) // (NS * CHUNK)
    tiles_chunks = jnp.maximum(32, ((tiles_chunks + 31) // 32) * 32)
    j = jnp.arange(4 * EPAD, dtype=jnp.int32)
    g_of = j // EPAD
    r = j % EPAD
    eidx = jnp.where(r < counts[g_of], start[g_of] + r, E)
    lists = sorted_p[eidx].reshape(4, EPAD // 128, 128)
    cnts = jnp.repeat((tiles_chunks // 2)[:, None], L, axis=1)
    zrows = jnp.zeros((RPT, D), jnp.float32)
    hpad = jnp.zeros((HROWS - N, D), jnp.float32)

    h = x
    for (eps, W1, b1, W2, b2) in (
        (eps0, W1_0, b1_0, W2_0, b2_0),
        (eps1, W1_1, b1_1, W2_1, b2_1),
        (eps2, W1_2, b1_2, W2_2, b2_2),
    ):
        h2 = jnp.concatenate([h, hpad])
        out = _sc_aggregate(h2, lists, cnts, zrows)
        agg = jnp.concatenate([out[0, :HALF], out[1, : N - HALF]])
        scale = jnp.reshape(1.0 + eps, (1,)).astype(jnp.float32)
        h = _tc_mlp(h, agg, scale, W1, b1, W2, b2)
    return h


# speedup vs baseline: 14.8513x; 14.8513x over previous
import functools

import jax
import jax.numpy as jnp
from jax import lax
from jax.experimental import pallas as pl
from jax.experimental.pallas import tpu as pltpu
from jax.experimental.pallas import tpu_sc as plsc

N = 10000
D = 128
H_DIM = 256
E = 320000

NC, NS, L = 2, 16, 16
NW = NC * NS
HALF = 5120
JUNK = HALF
AGGH = 5248
RPT = AGGH // NS
HROWS = 2 * HALF
HSPT = HALF // NS
CHUNK = 64
CAPC = 160
CAP = CAPC * CHUNK
EPAD = NW * CAP
NR = 4
A = 3
S = 1
P = 2 * NR
U = P - S
IDX_BITS = 14
IDX_MASK = (1 << IDX_BITS) - 1
JUNKP = JUNK << IDX_BITS

_MESH = plsc.VectorSubcoreMesh(
    core_axis_name="c", subcore_axis_name="s", num_cores=NC, num_subcores=NS
)


def _sc_aggregate(h2, lists, cnts, zrows):
    @functools.partial(
        pl.kernel,
        mesh=_MESH,
        out_type=jax.ShapeDtypeStruct((NC, AGGH, D), jnp.float32),
        scratch_types=[
            pltpu.VMEM((CAP // 128, 128), jnp.int32),
            pltpu.VMEM((4, 16), jnp.int32),
            pltpu.VMEM((P, CHUNK), jnp.int32),
            pltpu.VMEM((P, CHUNK), jnp.int32),
            pltpu.VMEM((NR, CHUNK, D), jnp.float32),
            pltpu.VMEM_SHARED((HALF, D), jnp.float32),
            pltpu.VMEM_SHARED((AGGH, D), jnp.float32),
            pltpu.SemaphoreType.DMA,
            pltpu.SemaphoreType.DMA,
        ],
    )
    def body(h_hbm, lists_hbm, cnt_hbm, z_hbm, out_hbm,
             pk_v, cnt_v, sidx_v, didx_v, rows_v, h_sh, agg_sh, gsem, ssem):
        cid = lax.axis_index("c")
        sid = lax.axis_index("s")
        pltpu.sync_copy(z_hbm, agg_sh.at[pl.ds(sid * RPT, RPT)])
        pltpu.sync_copy(cnt_hbm, cnt_v)

        def unpack(c, k):
            slot = k % P
            for v in range(CHUNK // L):
                p = pk_v[c // 2, pl.ds((c % 2) * CHUNK + v * L, L)]
                sidx_v[slot, pl.ds(v * L, L)] = jnp.bitwise_and(p, IDX_MASK)
                didx_v[slot, pl.ds(v * L, L)] = jnp.bitwise_and(
                    lax.shift_right_logical(p, IDX_BITS), IDX_MASK
                )

        def step(c, k, wait_scat, do_unpack, do_gather):
            b = k % NR
            pltpu.make_async_copy(h_sh.at[sidx_v.at[k % P]], rows_v.at[b], gsem).wait()
            pltpu.async_copy(rows_v.at[b], agg_sh.at[didx_v.at[k % P]], ssem, add=True)
            if wait_scat:
                pltpu.make_async_copy(
                    rows_v.at[(k - S) % NR], agg_sh.at[didx_v.at[(k - S) % P]], ssem
                ).wait()
            if do_unpack:
                unpack(c + U, k + U)
            if do_gather:
                pltpu.async_copy(
                    h_sh.at[sidx_v.at[(k + A) % P]], rows_v.at[(k + A) % NR], gsem
                )

        def run_list(tc):
            for c in range(U):
                unpack(c, c)
            for b in range(A):
                pltpu.async_copy(h_sh.at[sidx_v.at[b]], rows_v.at[b], gsem)
            for k in range(P):
                step(k, k, k >= S, True, True)

            @pl.loop(P, tc - P, step=P)
            def _(g):
                for k in range(P):
                    step(g + k, k, True, True, True)

            base_c = tc - P
            for k in range(P):
                step(base_c + k, k, True, k < S, k < P - A)
            for k in range(P - S, P):
                pltpu.make_async_copy(
                    rows_v.at[k % NR], agg_sh.at[didx_v.at[k]], ssem
                ).wait()

        for s in (0, 1):
            pltpu.sync_copy(
                h_hbm.at[pl.ds(s * HALF + sid * HSPT, HSPT)],
                h_sh.at[pl.ds(sid * HSPT, HSPT)],
            )
            plsc.subcore_barrier()
            g = s * 2 + cid
            tcv = cnt_v[g, pl.ds(0, L)]
            tch = tcv[0]
            for li in (0, 1):
                row0 = pl.multiple_of(
                    sid * tch + li * lax.shift_right_logical(tch, 1), 8
                )
                pltpu.sync_copy(
                    lists_hbm.at[g].at[pl.ds(row0, CAP // 128)], pk_v
                )
                run_list(tch)
            plsc.subcore_barrier()

        pltpu.sync_copy(
            agg_sh.at[pl.ds(sid * RPT, RPT)],
            out_hbm.at[cid].at[pl.ds(sid * RPT, RPT)],
        )

    return body(h2, lists, cnts, zrows)


def _mlp_block(scale_ref, h_ref, a_ref, w1_ref, b1_ref, w2_ref, b2_ref, o_ref):
    z = scale_ref[0] * h_ref[...] + a_ref[...]
    z = jnp.maximum(
        jnp.dot(z, w1_ref[...], preferred_element_type=jnp.float32) + b1_ref[...], 0.0
    )
    z = jnp.dot(z, w2_ref[...], preferred_element_type=jnp.float32) + b2_ref[...]
    o_ref[...] = jnp.maximum(z, 0.0)


def _tc_mlp(h, a, scale, W1, b1, W2, b2):
    R = 1000
    grid = (N // R,)
    return pl.pallas_call(
        _mlp_block,
        grid=grid,
        in_specs=[
            pl.BlockSpec(memory_space=pltpu.SMEM),
            pl.BlockSpec((R, D), lambda i: (i, 0)),
            pl.BlockSpec((R, D), lambda i: (i, 0)),
            pl.BlockSpec((D, H_DIM), lambda i: (0, 0)),
            pl.BlockSpec((1, H_DIM), lambda i: (0, 0)),
            pl.BlockSpec((H_DIM, D), lambda i: (0, 0)),
            pl.BlockSpec((1, D), lambda i: (0, 0)),
        ],
        out_specs=pl.BlockSpec((R, D), lambda i: (i, 0)),
        out_shape=jax.ShapeDtypeStruct((N, D), jnp.float32),
    )(scale, h, a, W1, b1.reshape(1, H_DIM), W2, b2.reshape(1, D))


def kernel(x, edge_index, batch,
           eps0, W1_0, b1_0, W2_0, b2_0,
           eps1, W1_1, b1_1, W2_1, b2_1,
           eps2, W1_2, b1_2, W2_2, b2_2):
    src = edge_index[0]
    dst = edge_index[1]
    s_bit = (src >= HALF).astype(jnp.int32)
    d_bit = (dst >= HALF).astype(jnp.int32)
    p = (
        lax.shift_left(s_bit, 2 * IDX_BITS + 1)
        | lax.shift_left(d_bit, 2 * IDX_BITS)
        | lax.shift_left(dst - d_bit * HALF, IDX_BITS)
        | (src - s_bit * HALF)
    )
    grp = s_bit * 2 + d_bit
    sorted_p = jnp.sort(p)
    counts = jnp.sum(
        grp[None, :] == jnp.arange(4, dtype=jnp.int32)[:, None], axis=1
    ).astype(jnp.int32)
    start = jnp.concatenate(
        [jnp.zeros((1,), jnp.int32), jnp.cumsum(counts)[:-1].astype(jnp.int32)]
    )
    tiles_chunks = (counts + (NS * CHUNK - 1)) // (NS * CHUNK)
    tiles_chunks = jnp.maximum(32, ((tiles_chunks + 31) // 32) * 32)
    sorted_pad = jnp.concatenate([sorted_p, jnp.full((EPAD,), JUNKP, jnp.int32)])
    r = jnp.arange(EPAD, dtype=jnp.int32)
    regions = [
        jnp.where(
            r < counts[g],
            lax.dynamic_slice(sorted_pad, (start[g],), (EPAD,)),
            JUNKP,
        )
        for g in range(4)
    ]
    lists = jnp.stack(regions).reshape(4, EPAD // 128, 128)
    cnts = jnp.repeat((tiles_chunks // 2)[:, None], L, axis=1)
    zrows = jnp.zeros((RPT, D), jnp.float32)
    hpad = jnp.zeros((HROWS - N, D), jnp.float32)

    h = x
    for (eps, W1, b1, W2, b2) in (
        (eps0, W1_0, b1_0, W2_0, b2_0),
        (eps1, W1_1, b1_1, W2_1, b2_1),
        (eps2, W1_2, b1_2, W2_2, b2_2),
    ):
        h2 = jnp.concatenate([h, hpad])
        out = _sc_aggregate(h2, lists, cnts, zrows)
        agg = jnp.concatenate([out[0, :HALF], out[1, : N - HALF]])
        scale = jnp.reshape(1.0 + eps, (1,)).astype(jnp.float32)
        h = _tc_mlp(h, agg, scale, W1, b1, W2, b2)
    return h


# speedup vs baseline: 19.2458x; 1.2959x over previous
import functools

import jax
import jax.numpy as jnp
from jax import lax
from jax.experimental import pallas as pl
from jax.experimental.pallas import tpu as pltpu
from jax.experimental.pallas import tpu_sc as plsc

N = 10000
D = 128
H_DIM = 256
E = 320000

NC, NS, L = 2, 16, 16
NW = NC * NS
HALF = 5120
JUNK = HALF
AGGH = 5248
RPT = AGGH // NS
HROWS = 2 * HALF
HSPT = HALF // NS
CHUNK = 64
CAPC = 160
CAP = CAPC * CHUNK
EPAD = NW * CAP
NR = 4
A = 3
S = 1
P = 2 * NR
U = P - S
IDX_BITS = 14
IDX_MASK = (1 << IDX_BITS) - 1
JUNKP = JUNK << IDX_BITS

_MESH = plsc.VectorSubcoreMesh(
    core_axis_name="c", subcore_axis_name="s", num_cores=NC, num_subcores=NS
)


def _sc_aggregate(h2, lists, cnts, zrows):
    @functools.partial(
        pl.kernel,
        mesh=_MESH,
        out_type=jax.ShapeDtypeStruct((NC, AGGH, D), jnp.float32),
        scratch_types=[
            pltpu.VMEM((CAP // 128, 128), jnp.int32),
            pltpu.VMEM((4, 16), jnp.int32),
            pltpu.VMEM((P, CHUNK), jnp.int32),
            pltpu.VMEM((P, CHUNK), jnp.int32),
            pltpu.VMEM((NR, CHUNK, D), jnp.float32),
            pltpu.VMEM_SHARED((HALF, D), jnp.float32),
            pltpu.VMEM_SHARED((AGGH, D), jnp.float32),
            pltpu.SemaphoreType.DMA,
            pltpu.SemaphoreType.DMA,
        ],
    )
    def body(h_hbm, lists_hbm, cnt_hbm, z_hbm, out_hbm,
             pk_v, cnt_v, sidx_v, didx_v, rows_v, h_sh, agg_sh, gsem, ssem):
        cid = lax.axis_index("c")
        sid = lax.axis_index("s")
        pltpu.sync_copy(z_hbm, agg_sh.at[pl.ds(sid * RPT, RPT)])
        pltpu.sync_copy(cnt_hbm, cnt_v)

        def unpack(c, k):
            slot = k % P
            for v in range(CHUNK // L):
                p = pk_v[c // 2, pl.ds((c % 2) * CHUNK + v * L, L)]
                sidx_v[slot, pl.ds(v * L, L)] = jnp.bitwise_and(p, IDX_MASK)
                didx_v[slot, pl.ds(v * L, L)] = jnp.bitwise_and(
                    lax.shift_right_logical(p, IDX_BITS), IDX_MASK
                )

        def step(c, k, wait_scat, do_unpack, do_gather):
            b = k % NR
            pltpu.make_async_copy(h_sh.at[sidx_v.at[k % P]], rows_v.at[b], gsem).wait()
            pltpu.async_copy(rows_v.at[b], agg_sh.at[didx_v.at[k % P]], ssem, add=True)
            if wait_scat:
                pltpu.make_async_copy(
                    rows_v.at[(k - S) % NR], agg_sh.at[didx_v.at[(k - S) % P]], ssem
                ).wait()
            if do_unpack:
                unpack(c + U, k + U)
            if do_gather:
                pltpu.async_copy(
                    h_sh.at[sidx_v.at[(k + A) % P]], rows_v.at[(k + A) % NR], gsem
                )

        def run_list(tc):
            for c in range(U):
                unpack(c, c)
            for b in range(A):
                pltpu.async_copy(h_sh.at[sidx_v.at[b]], rows_v.at[b], gsem)
            for k in range(P):
                step(k, k, k >= S, True, True)

            @pl.loop(P, tc - P, step=P)
            def _(g):
                for k in range(P):
                    step(g + k, k, True, True, True)

            base_c = tc - P
            for k in range(P):
                step(base_c + k, k, True, k < S, k < P - A)
            for k in range(P - S, P):
                pltpu.make_async_copy(
                    rows_v.at[k % NR], agg_sh.at[didx_v.at[k]], ssem
                ).wait()

        for s in (0, 1):
            pltpu.sync_copy(
                h_hbm.at[pl.ds(s * HALF + sid * HSPT, HSPT)],
                h_sh.at[pl.ds(sid * HSPT, HSPT)],
            )
            plsc.subcore_barrier()
            g = s * 2 + cid
            tcv = cnt_v[g, pl.ds(0, L)]
            tch = tcv[0]
            for li in (0, 1):
                row0 = pl.multiple_of(
                    sid * tch + li * lax.shift_right_logical(tch, 1), 8
                )
                pltpu.sync_copy(
                    lists_hbm.at[g].at[pl.ds(row0, CAP // 128)], pk_v
                )
                run_list(tch)
            plsc.subcore_barrier()

        pltpu.sync_copy(
            agg_sh.at[pl.ds(sid * RPT, RPT)],
            out_hbm.at[cid].at[pl.ds(sid * RPT, RPT)],
        )

    return body(h2, lists, cnts, zrows)


def _mlp_block(scale_ref, h_ref, a_ref, w1_ref, b1_ref, w2_ref, b2_ref, o_ref):
    z = scale_ref[0] * h_ref[...] + a_ref[...]
    z = jnp.maximum(
        jnp.dot(z, w1_ref[...], preferred_element_type=jnp.float32) + b1_ref[...], 0.0
    )
    z = jnp.dot(z, w2_ref[...], preferred_element_type=jnp.float32) + b2_ref[...]
    o_ref[...] = jnp.maximum(z, 0.0)


def _tc_mlp(h, a, scale, W1, b1, W2, b2):
    R = 1000
    grid = (N // R,)
    return pl.pallas_call(
        _mlp_block,
        grid=grid,
        in_specs=[
            pl.BlockSpec(memory_space=pltpu.SMEM),
            pl.BlockSpec((R, D), lambda i: (i, 0)),
            pl.BlockSpec((R, D), lambda i: (i, 0)),
            pl.BlockSpec((D, H_DIM), lambda i: (0, 0)),
            pl.BlockSpec((1, H_DIM), lambda i: (0, 0)),
            pl.BlockSpec((H_DIM, D), lambda i: (0, 0)),
            pl.BlockSpec((1, D), lambda i: (0, 0)),
        ],
        out_specs=pl.BlockSpec((R, D), lambda i: (i, 0)),
        out_shape=jax.ShapeDtypeStruct((N, D), jnp.float32),
    )(scale, h, a, W1, b1.reshape(1, H_DIM), W2, b2.reshape(1, D))


def kernel(x, edge_index, batch,
           eps0, W1_0, b1_0, W2_0, b2_0,
           eps1, W1_1, b1_1, W2_1, b2_1,
           eps2, W1_2, b1_2, W2_2, b2_2):
    src = edge_index[0]
    dst = edge_index[1]
    s_bit = (src >= HALF).astype(jnp.int32)
    d_bit = (dst >= HALF).astype(jnp.int32)
    p = (
        lax.shift_left(s_bit, 2 * IDX_BITS + 1)
        | lax.shift_left(d_bit, 2 * IDX_BITS)
        | lax.shift_left(dst - d_bit * HALF, IDX_BITS)
        | (src - s_bit * HALF)
    )
    grp = s_bit * 2 + d_bit
    sorted_p = lax.sort(p, is_stable=False)
    counts = jnp.sum(
        grp[None, :] == jnp.arange(4, dtype=jnp.int32)[:, None], axis=1
    ).astype(jnp.int32)
    start = jnp.concatenate(
        [jnp.zeros((1,), jnp.int32), jnp.cumsum(counts)[:-1].astype(jnp.int32)]
    )
    tiles_chunks = (counts + (NS * CHUNK - 1)) // (NS * CHUNK)
    tiles_chunks = jnp.maximum(32, ((tiles_chunks + 31) // 32) * 32)
    sorted_pad = jnp.concatenate([sorted_p, jnp.full((EPAD,), JUNKP, jnp.int32)])
    r = jnp.arange(EPAD, dtype=jnp.int32)
    regions = [
        jnp.where(
            r < counts[g],
            lax.dynamic_slice(sorted_pad, (start[g],), (EPAD,)),
            JUNKP,
        )
        for g in range(4)
    ]
    lists = jnp.stack(regions).reshape(4, EPAD // 128, 128)
    cnts = jnp.repeat((tiles_chunks // 2)[:, None], L, axis=1)
    zrows = jnp.zeros((RPT, D), jnp.float32)
    hpad = jnp.zeros((HROWS - N, D), jnp.float32)

    h = x
    for (eps, W1, b1, W2, b2) in (
        (eps0, W1_0, b1_0, W2_0, b2_0),
        (eps1, W1_1, b1_1, W2_1, b2_1),
        (eps2, W1_2, b1_2, W2_2, b2_2),
    ):
        h2 = jnp.concatenate([h, hpad])
        out = _sc_aggregate(h2, lists, cnts, zrows)
        agg = jnp.concatenate([out[0, :HALF], out[1, : N - HALF]])
        scale = jnp.reshape(1.0 + eps, (1,)).astype(jnp.float32)
        h = _tc_mlp(h, agg, scale, W1, b1, W2, b2)
    return h
